# baseline (speedup 1.0000x reference)
import functools

import jax
import jax.numpy as jnp
from jax import lax
from jax.experimental import pallas as pl
from jax.experimental.pallas import tpu as pltpu
from jax.experimental.pallas import tpu_sc as plsc

N_NODES = 100000
HIDDEN = 128
LANES = 16
NW = 32
IDX_MINOR = 100
IDX_ROWS = 2
CHUNK = IDX_ROWS * IDX_MINOR
STEPS = 16
SPAN_IDX_ROWS = STEPS * IDX_ROWS
NPAIRS = STEPS // 2


@functools.partial(
    pl.kernel,
    out_type=jax.ShapeDtypeStruct((N_NODES, HIDDEN), jnp.float32),
    mesh=plsc.VectorSubcoreMesh(core_axis_name="c", subcore_axis_name="s"),
    scratch_types=[
        pltpu.VMEM((SPAN_IDX_ROWS, IDX_MINOR), jnp.int32),
        pltpu.VMEM((CHUNK, HIDDEN), jnp.float32),
        pltpu.VMEM((CHUNK, HIDDEN), jnp.float32),
        pltpu.VMEM((CHUNK, HIDDEN), jnp.float32),
        pltpu.VMEM((CHUNK, HIDDEN), jnp.float32),
        pltpu.VMEM((2 * LANES,), jnp.float32),
        pltpu.SemaphoreType.DMA,
        pltpu.SemaphoreType.DMA,
        pltpu.SemaphoreType.DMA,
        pltpu.SemaphoreType.DMA,
    ],
)
def _sc_fused(feat_hbm, idx_hbm, pe_hbm, ab_hbm, out_hbm,
              idx_v, feat_a, feat_b, pe_a, pe_b, ab_v,
              sem_la, sem_lb, sem_wa, sem_wb):
    wid = lax.axis_index("s") * 2 + lax.axis_index("c")

    @pl.when(wid < NW - 1)
    def _():
        pltpu.sync_copy(idx_hbm.at[pl.ds(wid * SPAN_IDX_ROWS,
                                         SPAN_IDX_ROWS)], idx_v)

    @pl.when(wid == NW - 1)
    def _():
        pltpu.sync_copy(idx_hbm.at[pl.ds(wid * SPAN_IDX_ROWS, 8)],
                        idx_v.at[pl.ds(0, 8)])
    pltpu.sync_copy(ab_hbm, ab_v)
    va = ab_v[pl.ds(0, LANES)]
    vb = ab_v[pl.ds(LANES, LANES)]
    n_w = jnp.where(wid < NW - 1, STEPS, 4)

    def start_load(i, feat_v, pe_v, sem_l):
        base = (wid * STEPS + i) * CHUNK
        pltpu.async_copy(feat_hbm.at[pl.ds(base, CHUNK)], feat_v, sem_l)
        for k in range(IDX_ROWS):
            pltpu.async_copy(pe_hbm.at[idx_v.at[i * IDX_ROWS + k]],
                             pe_v.at[pl.ds(k * IDX_MINOR, IDX_MINOR)], sem_l)

    def wait_load(feat_v, pe_v, sem_l):
        pltpu.make_async_copy(feat_hbm.at[pl.ds(0, CHUNK)], feat_v,
                              sem_l).wait()
        pltpu.make_async_copy(feat_hbm.at[pl.ds(0, CHUNK)], pe_v,
                              sem_l).wait()

    def wait_wb(feat_v, sem_w):
        pltpu.make_async_copy(feat_v, out_hbm.at[pl.ds(0, CHUNK)],
                              sem_w).wait()

    def compute_store(i, feat_v, pe_v, sem_w):
        def row_body(r, rc):
            for k in range(HIDDEN // LANES):
                sl = pl.ds(k * LANES, LANES)
                feat_v[r, sl] = va * feat_v[r, sl] + vb * pe_v[r, sl]
            return rc

        lax.fori_loop(0, CHUNK, row_body, 0)
        base = (wid * STEPS + i) * CHUNK
        pltpu.async_copy(feat_v, out_hbm.at[pl.ds(base, CHUNK)], sem_w)

    start_load(0, feat_a, pe_a, sem_la)

    def pair_body(p, carry):
        i1 = 2 * p + 1
        i2 = 2 * p + 2

        wait_load(feat_a, pe_a, sem_la)

        @pl.when(p > 0)
        def _():
            wait_wb(feat_b, sem_wb)

        start_load(i1, feat_b, pe_b, sem_lb)
        compute_store(2 * p, feat_a, pe_a, sem_wa)
        wait_load(feat_b, pe_b, sem_lb)

        @pl.when(i2 < n_w)
        def _():
            wait_wb(feat_a, sem_wa)
            start_load(i2, feat_a, pe_a, sem_la)

        compute_store(i1, feat_b, pe_b, sem_wb)
        return carry

    lax.fori_loop(0, n_w // 2, pair_body, 0)

    wait_wb(feat_a, sem_wa)
    wait_wb(feat_b, sem_wb)


def kernel(bayesian_features, node_indices, pe_g, pe_m, pe_d, alpha, beta):
    idx2d = node_indices.astype(jnp.int32).reshape(
        N_NODES // IDX_MINOR, IDX_MINOR)
    ab = jnp.concatenate([
        jnp.broadcast_to(alpha.astype(jnp.float32), (LANES,)),
        jnp.broadcast_to(beta.astype(jnp.float32), (LANES,)),
    ])
    return _sc_fused(bayesian_features, idx2d, pe_g, ab)


# speedup vs baseline: 1.0090x; 1.0090x over previous
import functools

import jax
import jax.numpy as jnp
from jax import lax
from jax.experimental import pallas as pl
from jax.experimental.pallas import tpu as pltpu
from jax.experimental.pallas import tpu_sc as plsc

N_NODES = 100000
HIDDEN = 128
LANES = 16
NW = 32
IDX_MINOR = 100
IDX_ROWS = 2
CHUNK = IDX_ROWS * IDX_MINOR
STEPS = 16
SPAN_IDX_ROWS = STEPS * IDX_ROWS
NPAIRS = STEPS // 2


@functools.partial(
    pl.kernel,
    out_type=jax.ShapeDtypeStruct((N_NODES, HIDDEN), jnp.float32),
    mesh=plsc.VectorSubcoreMesh(core_axis_name="c", subcore_axis_name="s"),
    scratch_types=[
        pltpu.VMEM((SPAN_IDX_ROWS + 8, IDX_MINOR), jnp.int32),
        pltpu.VMEM((CHUNK, HIDDEN), jnp.float32),
        pltpu.VMEM((CHUNK, HIDDEN), jnp.float32),
        pltpu.VMEM((CHUNK, HIDDEN), jnp.float32),
        pltpu.VMEM((CHUNK, HIDDEN), jnp.float32),
        pltpu.VMEM((2 * LANES,), jnp.float32),
        pltpu.SemaphoreType.DMA,
        pltpu.SemaphoreType.DMA,
        pltpu.SemaphoreType.DMA,
        pltpu.SemaphoreType.DMA,
    ],
)
def _sc_fused(feat_hbm, idx_hbm, pe_hbm, ab_hbm, out_hbm,
              idx_v, feat_a, feat_b, pe_a, pe_b, ab_v,
              sem_la, sem_lb, sem_wa, sem_wb):
    cid = lax.axis_index("c")
    sid = lax.axis_index("s")
    row_base = cid * (N_NODES // 2) + sid * (STEPS * CHUNK)
    irow_base = cid * (N_NODES // 2 // IDX_MINOR) + sid * SPAN_IDX_ROWS
    off = 4 * cid
    aligned_base = pl.multiple_of(irow_base - off, 8)
    n_w = jnp.where(sid < 15, STEPS, 10)

    @pl.when(sid < 15)
    def _():
        pltpu.sync_copy(idx_hbm.at[pl.ds(aligned_base,
                                         SPAN_IDX_ROWS + 8)], idx_v)

    @pl.when(sid == 15)
    def _():
        pltpu.sync_copy(idx_hbm.at[pl.ds(aligned_base, 24)],
                        idx_v.at[pl.ds(0, 24)])
    pltpu.sync_copy(ab_hbm, ab_v)
    va = ab_v[pl.ds(0, LANES)]
    vb = ab_v[pl.ds(LANES, LANES)]

    def start_load(i, feat_v, pe_v, sem_l):
        base = row_base + i * CHUNK
        pltpu.async_copy(feat_hbm.at[pl.ds(base, CHUNK)], feat_v, sem_l)
        for k in range(IDX_ROWS):
            pltpu.async_copy(pe_hbm.at[idx_v.at[off + i * IDX_ROWS + k]],
                             pe_v.at[pl.ds(k * IDX_MINOR, IDX_MINOR)], sem_l)

    def wait_load(feat_v, pe_v, sem_l):
        pltpu.make_async_copy(feat_hbm.at[pl.ds(0, CHUNK)], feat_v,
                              sem_l).wait()
        pltpu.make_async_copy(feat_hbm.at[pl.ds(0, CHUNK)], pe_v,
                              sem_l).wait()

    def wait_wb(feat_v, sem_w):
        pltpu.make_async_copy(feat_v, out_hbm.at[pl.ds(0, CHUNK)],
                              sem_w).wait()

    def compute_store(i, feat_v, pe_v, sem_w):
        def row_body(r, rc):
            for k in range(HIDDEN // LANES):
                sl = pl.ds(k * LANES, LANES)
                feat_v[r, sl] = va * feat_v[r, sl] + vb * pe_v[r, sl]
            return rc

        lax.fori_loop(0, CHUNK, row_body, 0)
        base = row_base + i * CHUNK
        pltpu.async_copy(feat_v, out_hbm.at[pl.ds(base, CHUNK)], sem_w)

    start_load(0, feat_a, pe_a, sem_la)

    def pair_body(p, carry):
        i1 = 2 * p + 1
        i2 = 2 * p + 2

        wait_load(feat_a, pe_a, sem_la)

        @pl.when(p > 0)
        def _():
            wait_wb(feat_b, sem_wb)

        start_load(i1, feat_b, pe_b, sem_lb)
        compute_store(2 * p, feat_a, pe_a, sem_wa)
        wait_load(feat_b, pe_b, sem_lb)

        @pl.when(i2 < n_w)
        def _():
            wait_wb(feat_a, sem_wa)
            start_load(i2, feat_a, pe_a, sem_la)

        compute_store(i1, feat_b, pe_b, sem_wb)
        return carry

    lax.fori_loop(0, n_w // 2, pair_body, 0)

    wait_wb(feat_a, sem_wa)
    wait_wb(feat_b, sem_wb)


def kernel(bayesian_features, node_indices, pe_g, pe_m, pe_d, alpha, beta):
    idx2d = node_indices.astype(jnp.int32).reshape(
        N_NODES // IDX_MINOR, IDX_MINOR)
    ab = jnp.concatenate([
        jnp.broadcast_to(alpha.astype(jnp.float32), (LANES,)),
        jnp.broadcast_to(beta.astype(jnp.float32), (LANES,)),
    ])
    return _sc_fused(bayesian_features, idx2d, pe_g, ab)
